# baseline (speedup 1.0000x reference)
import functools

import jax
import jax.numpy as jnp
from jax import lax
from jax.experimental import pallas as pl
from jax.experimental.pallas import tpu as pltpu
from jax.experimental.pallas import tpu_sc as plsc

N = 10000
E = 320000
C = 128
G = 16
NC = 2
NS = 16
NW = NC * NS
NPAD = 10240
TPW = NPAD // NS
QTPW = 80
B = 125
RB = 8
EROWS = E // B
WROWS = EROWS // NW
PH = 2
PROWS = WROWS // PH
CHUNKS = EROWS // (NW * RB)
RN = 2000


def _worker_id():
    return lax.axis_index("c") * NS + lax.axis_index("s")


def _deg_body(dst2_hbm, zvec_hbm, ovec_hbm, out_hbm, dacc, didx, ones_v, bounce):
    cid = lax.axis_index("c")
    sid = lax.axis_index("s")
    wid = cid * NS + sid
    pltpu.sync_copy(zvec_hbm, bounce)
    pltpu.sync_copy(ovec_hbm, ones_v)
    pltpu.sync_copy(bounce, dacc.at[pl.ds(sid * TPW, TPW)])
    plsc.subcore_barrier()

    def chunk(ci, carry):
        row0 = (ci * NW + wid) * RB
        pltpu.sync_copy(dst2_hbm.at[pl.ds(row0, RB)], didx)
        for j in range(RB):
            pltpu.sync_copy(ones_v, dacc.at[didx.at[j]], add=True)
        return carry

    lax.fori_loop(0, CHUNKS, chunk, 0)
    plsc.subcore_barrier()
    pltpu.sync_copy(dacc.at[pl.ds(sid * TPW, TPW)], bounce)
    pltpu.sync_copy(bounce, out_hbm.at[pl.ds(cid * NPAD + sid * TPW, TPW)])


@functools.cache
def _deg_kernel():
    mesh = plsc.VectorSubcoreMesh(core_axis_name="c", subcore_axis_name="s",
                                  num_cores=NC, num_subcores=NS)
    return pl.kernel(
        _deg_body,
        out_type=jax.ShapeDtypeStruct((NC * NPAD,), jnp.float32),
        mesh=mesh,
        scratch_types=[
            pltpu.VMEM_SHARED((NPAD,), jnp.float32),
            pltpu.VMEM((RB, B), jnp.int32),
            pltpu.VMEM((B,), jnp.float32),
            pltpu.VMEM((TPW,), jnp.float32),
        ],
    )


def _scatter_body(xs_hbm, src2_hbm, dst2_hbm, zrows_hbm, out_hbm,
                  acc, sidx, didx, buf0, buf1, gs0, gs1, ss0, ss1):
    cid = lax.axis_index("c")
    sid = lax.axis_index("s")
    wid = cid * NS + sid
    r0 = sid * TPW
    bufs = (buf0, buf1)
    gsems = (gs0, gs1)
    ssems = (ss0, ss1)
    pltpu.sync_copy(zrows_hbm, buf0.at[pl.ds(0, QTPW)])
    for hh in range(TPW // QTPW):
        pltpu.sync_copy(buf0.at[pl.ds(0, QTPW)],
                        acc.at[pl.ds(r0 + hh * QTPW, QTPW)])
    plsc.subcore_barrier()

    for ph in range(PH):
        row0 = wid * WROWS + ph * PROWS
        pltpu.sync_copy(src2_hbm.at[pl.ds(row0, PROWS)], sidx)
        pltpu.sync_copy(dst2_hbm.at[pl.ds(row0, PROWS)], didx)

        pltpu.async_copy(xs_hbm.at[sidx.at[0]], buf0, gs0)

        def step(t2, carry):
            for p in range(2):
                t = 2 * t2 + p

                @pl.when(t >= 1)
                def _drain_prev():
                    pltpu.make_async_copy(bufs[1 - p],
                                          acc.at[didx.at[t - 1]],
                                          ssems[1 - p]).wait()

                @pl.when(t + 1 < PROWS)
                def _fire_next():
                    pltpu.async_copy(xs_hbm.at[sidx.at[t + 1]],
                                     bufs[1 - p], gsems[1 - p])
                pltpu.make_async_copy(xs_hbm.at[sidx.at[t]], bufs[p],
                                      gsems[p]).wait()
                pltpu.async_copy(bufs[p], acc.at[didx.at[t]],
                                 ssems[p], add=True)
            return carry

        lax.fori_loop(0, PROWS // 2, step, 0)
        lastp = (PROWS - 1) % 2
        pltpu.make_async_copy(bufs[lastp],
                              acc.at[didx.at[PROWS - 1]],
                              ssems[lastp]).wait()
    plsc.subcore_barrier()
    for hh in range(TPW // QTPW):
        pltpu.sync_copy(acc.at[pl.ds(r0 + hh * QTPW, QTPW)],
                        buf0.at[pl.ds(0, QTPW)])
        pltpu.sync_copy(buf0.at[pl.ds(0, QTPW)],
                        out_hbm.at[pl.ds(cid * NPAD + r0 + hh * QTPW, QTPW)])


@functools.cache
def _scatter_kernel():
    mesh = plsc.VectorSubcoreMesh(core_axis_name="c", subcore_axis_name="s",
                                  num_cores=NC, num_subcores=NS)
    return pl.kernel(
        _scatter_body,
        out_type=jax.ShapeDtypeStruct((NC * NPAD, C), jnp.float32),
        mesh=mesh,
        scratch_types=[
            pltpu.VMEM_SHARED((NPAD, C), jnp.float32),
            pltpu.VMEM((PROWS, B), jnp.int32),
            pltpu.VMEM((PROWS, B), jnp.int32),
            pltpu.VMEM((B, C), jnp.float32),
            pltpu.VMEM((B, C), jnp.float32),
            pltpu.SemaphoreType.DMA,
            pltpu.SemaphoreType.DMA,
            pltpu.SemaphoreType.DMA,
            pltpu.SemaphoreType.DMA,
        ],
    )


PREC_LAYER = None
PREC_POOL = "highest"
PREC_HEAD = None


def _leaky(h):
    return jnp.where(h > 0, h, 0.01 * h)


def _prep_body(da_ref, db_ref, x_ref, rdeg_ref, xs_ref):
    rd = lax.rsqrt(da_ref[...] + db_ref[...] + 1.0)
    rdeg_ref[...] = rd
    xs_ref[...] = x_ref[...] * rd


def _mid_body(sa_ref, sb_ref, xs_ref, rd_ref, w_ref, b_ref, xsn_ref):
    rd = rd_ref[...]
    agg = (sa_ref[...] + sb_ref[...] + xs_ref[...]) * rd
    h = jnp.dot(agg, w_ref[...], preferred_element_type=jnp.float32,
                precision=PREC_LAYER) + b_ref[...]
    xsn_ref[...] = _leaky(h) * rd


def _final_body(sa_ref, sb_ref, xs_ref, rd_ref, i_ref, w_ref, b_ref,
                wd_ref, bd_ref, wo_ref, bo_ref, out_ref, pooled, cnt):
    bi = pl.program_id(0)

    @pl.when(bi == 0)
    def _init():
        pooled[...] = jnp.zeros_like(pooled)
        cnt[...] = jnp.zeros_like(cnt)

    agg = (sa_ref[...] + sb_ref[...] + xs_ref[...]) * rd_ref[...]
    h = _leaky(jnp.dot(agg, w_ref[...], preferred_element_type=jnp.float32,
                       precision=PREC_LAYER) + b_ref[...])
    seg = lax.broadcasted_iota(jnp.int32, (RN, G), 1)
    oh = (seg == i_ref[...]).astype(jnp.float32)
    dn = (((0,), (0,)), ((), ()))
    pooled[...] += lax.dot_general(oh, h, dn,
                                   preferred_element_type=jnp.float32,
                                   precision=PREC_POOL)
    cnt[...] += lax.dot_general(oh, jnp.ones_like(h), dn,
                                preferred_element_type=jnp.float32,
                                precision=PREC_POOL)

    @pl.when(bi == pl.num_programs(0) - 1)
    def _head():
        pm = pooled[...] / jnp.maximum(cnt[...], 1.0)
        d = _leaky(jnp.dot(pm, wd_ref[...], preferred_element_type=jnp.float32,
                           precision=PREC_HEAD) + bd_ref[...])
        o = jnp.dot(d, wo_ref[...], preferred_element_type=jnp.float32,
                    precision=PREC_HEAD) + bo_ref[...]
        out_ref[...] = jax.nn.sigmoid(o)


_row = lambda b: (b, 0)
_whole = lambda b: (0, 0)

_prep_kernel = pl.pallas_call(
    _prep_body,
    grid=(N // RN,),
    in_specs=[
        pl.BlockSpec((RN, 1), _row),
        pl.BlockSpec((RN, 1), _row),
        pl.BlockSpec((RN, C), _row),
    ],
    out_specs=[
        pl.BlockSpec((RN, 1), _row),
        pl.BlockSpec((RN, C), _row),
    ],
    out_shape=[
        jax.ShapeDtypeStruct((N, 1), jnp.float32),
        jax.ShapeDtypeStruct((N, C), jnp.float32),
    ],
)

_mid_kernel = pl.pallas_call(
    _mid_body,
    grid=(N // RN,),
    in_specs=[
        pl.BlockSpec((RN, C), _row),
        pl.BlockSpec((RN, C), _row),
        pl.BlockSpec((RN, C), _row),
        pl.BlockSpec((RN, 1), _row),
        pl.BlockSpec((C, C), _whole),
        pl.BlockSpec((1, C), _whole),
    ],
    out_specs=pl.BlockSpec((RN, C), _row),
    out_shape=jax.ShapeDtypeStruct((N, C), jnp.float32),
)

_final_kernel = pl.pallas_call(
    _final_body,
    grid=(N // RN,),
    in_specs=[
        pl.BlockSpec((RN, C), _row),
        pl.BlockSpec((RN, C), _row),
        pl.BlockSpec((RN, C), _row),
        pl.BlockSpec((RN, 1), _row),
        pl.BlockSpec((RN, 1), _row),
        pl.BlockSpec((C, C), _whole),
        pl.BlockSpec((1, C), _whole),
        pl.BlockSpec((C, C), _whole),
        pl.BlockSpec((1, C), _whole),
        pl.BlockSpec((C, 1), _whole),
        pl.BlockSpec((1, 1), _whole),
    ],
    out_specs=pl.BlockSpec((G, 1), _whole),
    out_shape=jax.ShapeDtypeStruct((G, 1), jnp.float32),
    scratch_shapes=[
        pltpu.VMEM((G, C), jnp.float32),
        pltpu.VMEM((G, C), jnp.float32),
    ],
)


def kernel(x, edge_index, e, i, W1, b1, W2, b2, Wd, bd, Wo, bo):
    del e
    src2 = edge_index[0].reshape(EROWS, B)
    dst2 = edge_index[1].reshape(EROWS, B)
    zvec = jnp.zeros((TPW,), jnp.float32)
    ovec = jnp.ones((B,), jnp.float32)
    zrows = jnp.zeros((QTPW, C), jnp.float32)

    deg = _deg_kernel()(dst2, zvec, ovec)
    da = deg[:N].reshape(N, 1)
    db = deg[NPAD:NPAD + N].reshape(N, 1)
    rdeg, xs1 = _prep_kernel(da, db, x)

    s1 = _scatter_kernel()(xs1, src2, dst2, zrows)
    xs2 = _mid_kernel(s1[:N], s1[NPAD:NPAD + N], xs1, rdeg,
                      W1, b1.reshape(1, C))

    s2 = _scatter_kernel()(xs2, src2, dst2, zrows)
    out = _final_kernel(s2[:N], s2[NPAD:NPAD + N], xs2, rdeg,
                        i.reshape(N, 1), W2, b2.reshape(1, C),
                        Wd, bd.reshape(1, C), Wo.reshape(C, 1),
                        bo.reshape(1, 1))
    return out


# speedup vs baseline: 1.0038x; 1.0038x over previous
import functools

import jax
import jax.numpy as jnp
from jax import lax
from jax.experimental import pallas as pl
from jax.experimental.pallas import tpu as pltpu
from jax.experimental.pallas import tpu_sc as plsc

N = 10000
E = 320000
C = 128
G = 16
NC = 2
NS = 16
NW = NC * NS
NPAD = 10240
TPW = NPAD // NS
QTPW = 80
B = 125
RB = 8
EROWS = E // B
WROWS = EROWS // NW
PH = 2
PROWS = WROWS // PH
CHUNKS = EROWS // (NW * RB)
RN = 5000


def _worker_id():
    return lax.axis_index("c") * NS + lax.axis_index("s")


def _deg_body(dst2_hbm, zvec_hbm, ovec_hbm, out_hbm, dacc, didx, ones_v, bounce):
    cid = lax.axis_index("c")
    sid = lax.axis_index("s")
    wid = cid * NS + sid
    pltpu.sync_copy(zvec_hbm, bounce)
    pltpu.sync_copy(ovec_hbm, ones_v)
    pltpu.sync_copy(bounce, dacc.at[pl.ds(sid * TPW, TPW)])
    plsc.subcore_barrier()

    def chunk(ci, carry):
        row0 = (ci * NW + wid) * RB
        pltpu.sync_copy(dst2_hbm.at[pl.ds(row0, RB)], didx)
        for j in range(RB):
            pltpu.sync_copy(ones_v, dacc.at[didx.at[j]], add=True)
        return carry

    lax.fori_loop(0, CHUNKS, chunk, 0)
    plsc.subcore_barrier()
    pltpu.sync_copy(dacc.at[pl.ds(sid * TPW, TPW)], bounce)
    pltpu.sync_copy(bounce, out_hbm.at[pl.ds(cid * NPAD + sid * TPW, TPW)])


@functools.cache
def _deg_kernel():
    mesh = plsc.VectorSubcoreMesh(core_axis_name="c", subcore_axis_name="s",
                                  num_cores=NC, num_subcores=NS)
    return pl.kernel(
        _deg_body,
        out_type=jax.ShapeDtypeStruct((NC * NPAD,), jnp.float32),
        mesh=mesh,
        scratch_types=[
            pltpu.VMEM_SHARED((NPAD,), jnp.float32),
            pltpu.VMEM((RB, B), jnp.int32),
            pltpu.VMEM((B,), jnp.float32),
            pltpu.VMEM((TPW,), jnp.float32),
        ],
    )


def _scatter_body(xs_hbm, src2_hbm, dst2_hbm, zrows_hbm, out_hbm,
                  acc, sidx, didx, buf0, buf1, gs0, gs1, ss0, ss1):
    cid = lax.axis_index("c")
    sid = lax.axis_index("s")
    wid = cid * NS + sid
    r0 = sid * TPW
    bufs = (buf0, buf1)
    gsems = (gs0, gs1)
    ssems = (ss0, ss1)
    pltpu.sync_copy(zrows_hbm, buf0.at[pl.ds(0, QTPW)])
    for hh in range(TPW // QTPW):
        pltpu.sync_copy(buf0.at[pl.ds(0, QTPW)],
                        acc.at[pl.ds(r0 + hh * QTPW, QTPW)])
    plsc.subcore_barrier()

    for ph in range(PH):
        row0 = wid * WROWS + ph * PROWS
        pltpu.sync_copy(src2_hbm.at[pl.ds(row0, PROWS)], sidx)
        pltpu.sync_copy(dst2_hbm.at[pl.ds(row0, PROWS)], didx)

        pltpu.async_copy(xs_hbm.at[sidx.at[0]], buf0, gs0)

        def step(t2, carry):
            for p in range(2):
                t = 2 * t2 + p

                @pl.when(t >= 1)
                def _drain_prev():
                    pltpu.make_async_copy(bufs[1 - p],
                                          acc.at[didx.at[t - 1]],
                                          ssems[1 - p]).wait()

                @pl.when(t + 1 < PROWS)
                def _fire_next():
                    pltpu.async_copy(xs_hbm.at[sidx.at[t + 1]],
                                     bufs[1 - p], gsems[1 - p])
                pltpu.make_async_copy(xs_hbm.at[sidx.at[t]], bufs[p],
                                      gsems[p]).wait()
                pltpu.async_copy(bufs[p], acc.at[didx.at[t]],
                                 ssems[p], add=True)
            return carry

        lax.fori_loop(0, PROWS // 2, step, 0)
        lastp = (PROWS - 1) % 2
        pltpu.make_async_copy(bufs[lastp],
                              acc.at[didx.at[PROWS - 1]],
                              ssems[lastp]).wait()
    plsc.subcore_barrier()
    for hh in range(TPW // QTPW):
        pltpu.sync_copy(acc.at[pl.ds(r0 + hh * QTPW, QTPW)],
                        buf0.at[pl.ds(0, QTPW)])
        pltpu.sync_copy(buf0.at[pl.ds(0, QTPW)],
                        out_hbm.at[pl.ds(cid * NPAD + r0 + hh * QTPW, QTPW)])


@functools.cache
def _scatter_kernel():
    mesh = plsc.VectorSubcoreMesh(core_axis_name="c", subcore_axis_name="s",
                                  num_cores=NC, num_subcores=NS)
    return pl.kernel(
        _scatter_body,
        out_type=jax.ShapeDtypeStruct((NC * NPAD, C), jnp.float32),
        mesh=mesh,
        scratch_types=[
            pltpu.VMEM_SHARED((NPAD, C), jnp.float32),
            pltpu.VMEM((PROWS, B), jnp.int32),
            pltpu.VMEM((PROWS, B), jnp.int32),
            pltpu.VMEM((B, C), jnp.float32),
            pltpu.VMEM((B, C), jnp.float32),
            pltpu.SemaphoreType.DMA,
            pltpu.SemaphoreType.DMA,
            pltpu.SemaphoreType.DMA,
            pltpu.SemaphoreType.DMA,
        ],
    )


PREC_LAYER = None
PREC_POOL = "highest"
PREC_HEAD = None


def _leaky(h):
    return jnp.where(h > 0, h, 0.01 * h)


def _prep_body(da_ref, db_ref, x_ref, rdeg_ref, xs_ref):
    rd = lax.rsqrt(da_ref[...] + db_ref[...] + 1.0)
    rdeg_ref[...] = rd
    xs_ref[...] = x_ref[...] * rd


def _mid_body(sa_ref, sb_ref, xs_ref, rd_ref, w_ref, b_ref, xsn_ref):
    rd = rd_ref[...]
    agg = (sa_ref[...] + sb_ref[...] + xs_ref[...]) * rd
    h = jnp.dot(agg, w_ref[...], preferred_element_type=jnp.float32,
                precision=PREC_LAYER) + b_ref[...]
    xsn_ref[...] = _leaky(h) * rd


def _final_body(sa_ref, sb_ref, xs_ref, rd_ref, i_ref, w_ref, b_ref,
                wd_ref, bd_ref, wo_ref, bo_ref, out_ref, pooled, cnt):
    bi = pl.program_id(0)

    @pl.when(bi == 0)
    def _init():
        pooled[...] = jnp.zeros_like(pooled)
        cnt[...] = jnp.zeros_like(cnt)

    agg = (sa_ref[...] + sb_ref[...] + xs_ref[...]) * rd_ref[...]
    h = _leaky(jnp.dot(agg, w_ref[...], preferred_element_type=jnp.float32,
                       precision=PREC_LAYER) + b_ref[...])
    seg = lax.broadcasted_iota(jnp.int32, (RN, G), 1)
    oh = (seg == i_ref[...]).astype(jnp.float32)
    dn = (((0,), (0,)), ((), ()))
    pooled[...] += lax.dot_general(oh, h, dn,
                                   preferred_element_type=jnp.float32,
                                   precision=PREC_POOL)
    cnt[...] += lax.dot_general(oh, jnp.ones_like(h), dn,
                                preferred_element_type=jnp.float32,
                                precision=PREC_POOL)

    @pl.when(bi == pl.num_programs(0) - 1)
    def _head():
        pm = pooled[...] / jnp.maximum(cnt[...], 1.0)
        d = _leaky(jnp.dot(pm, wd_ref[...], preferred_element_type=jnp.float32,
                           precision=PREC_HEAD) + bd_ref[...])
        o = jnp.dot(d, wo_ref[...], preferred_element_type=jnp.float32,
                    precision=PREC_HEAD) + bo_ref[...]
        out_ref[...] = jax.nn.sigmoid(o)


_row = lambda b: (b, 0)
_whole = lambda b: (0, 0)

_prep_kernel = pl.pallas_call(
    _prep_body,
    grid=(N // RN,),
    in_specs=[
        pl.BlockSpec((RN, 1), _row),
        pl.BlockSpec((RN, 1), _row),
        pl.BlockSpec((RN, C), _row),
    ],
    out_specs=[
        pl.BlockSpec((RN, 1), _row),
        pl.BlockSpec((RN, C), _row),
    ],
    out_shape=[
        jax.ShapeDtypeStruct((N, 1), jnp.float32),
        jax.ShapeDtypeStruct((N, C), jnp.float32),
    ],
)

_mid_kernel = pl.pallas_call(
    _mid_body,
    grid=(N // RN,),
    in_specs=[
        pl.BlockSpec((RN, C), _row),
        pl.BlockSpec((RN, C), _row),
        pl.BlockSpec((RN, C), _row),
        pl.BlockSpec((RN, 1), _row),
        pl.BlockSpec((C, C), _whole),
        pl.BlockSpec((1, C), _whole),
    ],
    out_specs=pl.BlockSpec((RN, C), _row),
    out_shape=jax.ShapeDtypeStruct((N, C), jnp.float32),
)

_final_kernel = pl.pallas_call(
    _final_body,
    grid=(N // RN,),
    in_specs=[
        pl.BlockSpec((RN, C), _row),
        pl.BlockSpec((RN, C), _row),
        pl.BlockSpec((RN, C), _row),
        pl.BlockSpec((RN, 1), _row),
        pl.BlockSpec((RN, 1), _row),
        pl.BlockSpec((C, C), _whole),
        pl.BlockSpec((1, C), _whole),
        pl.BlockSpec((C, C), _whole),
        pl.BlockSpec((1, C), _whole),
        pl.BlockSpec((C, 1), _whole),
        pl.BlockSpec((1, 1), _whole),
    ],
    out_specs=pl.BlockSpec((G, 1), _whole),
    out_shape=jax.ShapeDtypeStruct((G, 1), jnp.float32),
    scratch_shapes=[
        pltpu.VMEM((G, C), jnp.float32),
        pltpu.VMEM((G, C), jnp.float32),
    ],
)


def kernel(x, edge_index, e, i, W1, b1, W2, b2, Wd, bd, Wo, bo):
    del e
    src2 = edge_index[0].reshape(EROWS, B)
    dst2 = edge_index[1].reshape(EROWS, B)
    zvec = jnp.zeros((TPW,), jnp.float32)
    ovec = jnp.ones((B,), jnp.float32)
    zrows = jnp.zeros((QTPW, C), jnp.float32)

    deg = _deg_kernel()(dst2, zvec, ovec)
    da = deg[:N].reshape(N, 1)
    db = deg[NPAD:NPAD + N].reshape(N, 1)
    rdeg, xs1 = _prep_kernel(da, db, x)

    s1 = _scatter_kernel()(xs1, src2, dst2, zrows)
    xs2 = _mid_kernel(s1[:N], s1[NPAD:NPAD + N], xs1, rdeg,
                      W1, b1.reshape(1, C))

    s2 = _scatter_kernel()(xs2, src2, dst2, zrows)
    out = _final_kernel(s2[:N], s2[NPAD:NPAD + N], xs2, rdeg,
                        i.reshape(N, 1), W2, b2.reshape(1, C),
                        Wd, bd.reshape(1, C), Wo.reshape(C, 1),
                        bo.reshape(1, 1))
    return out
